# Initial kernel scaffold; baseline (speedup 1.0000x reference)
import functools

import jax
import jax.numpy as jnp
from jax import lax
from jax.experimental import pallas as pl
from jax.experimental.pallas import tpu as pltpu
from jax.experimental.pallas import tpu_sc as plsc

N = 10000
E = 320000
D = 128
DP = 144
NC = 2
NS = 16
NW = NC * NS
EW = E // NW
B = 80
NCHUNK = EW // B
STRIPE = N // NS
ZR = 125


def _prep_body(x_ref, ws_ref, wd_ref, r0_ref, up_ref, v_ref, s0_ref):
    x = x_ref[...]
    u = jnp.dot(x, ws_ref[...], precision=lax.Precision.HIGHEST,
                preferred_element_type=jnp.float32)
    v_ref[...] = jnp.dot(x, wd_ref[...], precision=lax.Precision.HIGHEST,
                         preferred_element_type=jnp.float32)
    s0_ref[...] = jnp.dot(x, r0_ref[...], precision=lax.Precision.HIGHEST,
                          preferred_element_type=jnp.float32)
    up_ref[:, :D] = u
    r = up_ref.shape[0]
    col = lax.broadcasted_iota(jnp.int32, (r, DP - D), 1)
    up_ref[:, D:] = jnp.where(col == 0, 1.0, 0.0).astype(jnp.float32)


def _final_body(acc_ref, s0_ref, bias_ref, r1_ref, rb_ref, out_ref):
    a = acc_ref[0] + acc_ref[1]
    num = a[:, :D]
    den = a[:, D:D + 1]
    h = num / (den + 1e-9)
    f1 = h + bias_ref[...]
    f1 = jnp.maximum(f1, 0.2 * f1)
    out_ref[...] = (s0_ref[...] + rb_ref[...]
                    + jnp.dot(f1, r1_ref[...], precision=lax.Precision.HIGHEST,
                              preferred_element_type=jnp.float32))


def _sc_edge_body(up_hbm, v_hbm, src_hbm, dst_hbm, attn_hbm, out_hbm,
                  idx_src, idx_dst, u_buf, v_buf, attn_v, zbuf, acc_sh, sem):
    cid = lax.axis_index("c")
    sid = lax.axis_index("s")
    wid = cid * NS + sid
    zero16 = jnp.zeros((16,), jnp.float32)

    def zrow(r, c):
        for k in range(DP // 16):
            zbuf[r, pl.ds(k * 16, 16)] = zero16
        return c
    lax.fori_loop(0, ZR, zrow, 0)
    for t in range(STRIPE // ZR):
        pltpu.sync_copy(zbuf, acc_sh.at[pl.ds(sid * STRIPE + t * ZR, ZR)])
    plsc.subcore_barrier()

    pltpu.sync_copy(attn_hbm, attn_v)
    a_sl = tuple(attn_v[pl.ds(k * 16, 16)] for k in range(D // 16))

    def edge_body(e, c):
        acc = zero16
        for k in range(D // 16):
            uu = u_buf[e, pl.ds(k * 16, 16)]
            vv = v_buf[e, pl.ds(k * 16, 16)]
            x = uu + vv
            t = jnp.maximum(x, 0.2 * x)
            acc = acc + t * a_sl[k]
        s = jnp.sum(acc)
        wv = jnp.exp(s + zero16)
        for k in range(DP // 16):
            u_buf[e, pl.ds(k * 16, 16)] = u_buf[e, pl.ds(k * 16, 16)] * wv
        return c

    def chunk_body(c, carry):
        base = wid * EW + c * B
        pltpu.sync_copy(src_hbm.at[pl.ds(base, B)], idx_src)
        pltpu.sync_copy(dst_hbm.at[pl.ds(base, B)], idx_dst)
        pltpu.async_copy(up_hbm.at[idx_src], u_buf, sem).wait()
        pltpu.async_copy(v_hbm.at[idx_dst], v_buf, sem).wait()
        lax.fori_loop(0, B, edge_body, 0)
        pltpu.sync_copy(u_buf, acc_sh.at[idx_dst], add=True)
        return carry

    lax.fori_loop(0, NCHUNK, chunk_body, 0)
    plsc.subcore_barrier()
    pltpu.sync_copy(acc_sh.at[pl.ds(sid * STRIPE, STRIPE)],
                    out_hbm.at[cid, pl.ds(sid * STRIPE, STRIPE)])


_sc_edge = pl.kernel(
    _sc_edge_body,
    out_type=jax.ShapeDtypeStruct((NC, N, DP), jnp.float32),
    mesh=plsc.VectorSubcoreMesh(core_axis_name="c", subcore_axis_name="s",
                                num_cores=NC, num_subcores=NS),
    scratch_types=[
        pltpu.VMEM((B,), jnp.int32),
        pltpu.VMEM((B,), jnp.int32),
        pltpu.VMEM((B, DP), jnp.float32),
        pltpu.VMEM((B, D), jnp.float32),
        pltpu.VMEM((D,), jnp.float32),
        pltpu.VMEM((ZR, DP), jnp.float32),
        pltpu.VMEM_SHARED((N, DP), jnp.float32),
        pltpu.SemaphoreType.DMA,
    ],
)

_RB = 1000


def kernel(feats, edge_index, W_src, W_dst, attn, bias, R0_W, R0_b, R1_W, R1_b):
    src = edge_index[0].astype(jnp.int32)
    dst = edge_index[1].astype(jnp.int32)
    attn_flat = attn.reshape(D).astype(jnp.float32)

    up, v, s0 = pl.pallas_call(
        _prep_body,
        grid=(N // _RB,),
        in_specs=[
            pl.BlockSpec((_RB, D), lambda i: (i, 0)),
            pl.BlockSpec((D, D), lambda i: (0, 0)),
            pl.BlockSpec((D, D), lambda i: (0, 0)),
            pl.BlockSpec((D, D), lambda i: (0, 0)),
        ],
        out_specs=[
            pl.BlockSpec((_RB, DP), lambda i: (i, 0)),
            pl.BlockSpec((_RB, D), lambda i: (i, 0)),
            pl.BlockSpec((_RB, D), lambda i: (i, 0)),
        ],
        out_shape=[
            jax.ShapeDtypeStruct((N, DP), jnp.float32),
            jax.ShapeDtypeStruct((N, D), jnp.float32),
            jax.ShapeDtypeStruct((N, D), jnp.float32),
        ],
    )(feats, W_src, W_dst, R0_W)

    acc = _sc_edge(up, v, src, dst, attn_flat)

    rb = (R0_b + R1_b).reshape(1, D).astype(jnp.float32)
    bias2 = bias.reshape(1, D).astype(jnp.float32)
    out = pl.pallas_call(
        _final_body,
        grid=(N // _RB,),
        in_specs=[
            pl.BlockSpec((NC, _RB, DP), lambda i: (0, i, 0)),
            pl.BlockSpec((_RB, D), lambda i: (i, 0)),
            pl.BlockSpec((1, D), lambda i: (0, 0)),
            pl.BlockSpec((D, D), lambda i: (0, 0)),
            pl.BlockSpec((1, D), lambda i: (0, 0)),
        ],
        out_specs=pl.BlockSpec((_RB, D), lambda i: (i, 0)),
        out_shape=jax.ShapeDtypeStruct((N, D), jnp.float32),
    )(acc, s0, bias2, R1_W, rb)
    return out


# speedup vs baseline: 8.2325x; 8.2325x over previous
import functools

import jax
import jax.numpy as jnp
from jax import lax
from jax.experimental import pallas as pl
from jax.experimental.pallas import tpu as pltpu
from jax.experimental.pallas import tpu_sc as plsc

N = 10000
E = 320000
D = 128
DP = 144
NC = 2
NS = 16
NW = NC * NS
EW = E // NW
B = 80
NCHUNK = EW // B
NP = 10240
ACC_ROWS = 10368
STRIPE = ACC_ROWS // NS
ZR = 72


def _prep_body(x_ref, ws_ref, wd_ref, r0_ref, up_ref, v_ref, s0_ref):
    x = x_ref[...]
    u = jnp.dot(x, ws_ref[...], precision=lax.Precision.HIGHEST,
                preferred_element_type=jnp.float32)
    v_ref[...] = jnp.dot(x, wd_ref[...], precision=lax.Precision.HIGHEST,
                         preferred_element_type=jnp.float32)
    s0_ref[...] = jnp.dot(x, r0_ref[...], precision=lax.Precision.HIGHEST,
                          preferred_element_type=jnp.float32)
    up_ref[...] = u


def _final_body(acc_ref, s0_ref, den_ref, bias_ref, r1_ref, rb_ref, out_ref):
    num = acc_ref[0] + acc_ref[1]
    h = num / (den_ref[...] + 1e-9)
    f1 = h + bias_ref[...]
    f1 = jnp.maximum(f1, 0.2 * f1)
    out_ref[...] = (s0_ref[...] + rb_ref[...]
                    + jnp.dot(f1, r1_ref[...], precision=lax.Precision.HIGHEST,
                              preferred_element_type=jnp.float32))


def _sc_edge_body(up_hbm, v_hbm, src_hbm, dst_hbm, attn_hbm, out_hbm,
                  idx_src, idx_dst, u_buf, v_buf, wmat, den2d,
                  attn_v, zbuf, acc_sh, sem):
    cid = lax.axis_index("c")
    sid = lax.axis_index("s")
    wid = cid * NS + sid
    zero16 = jnp.zeros((16,), jnp.float32)
    iota16 = lax.iota(jnp.int32, 16)

    def zrow(r, c):
        for k in range(D // 16):
            zbuf[r, pl.ds(k * 16, 16)] = zero16
        return c
    lax.fori_loop(0, ZR, zrow, 0)
    for t in range(STRIPE // ZR):
        pltpu.sync_copy(zbuf, acc_sh.at[pl.ds(sid * STRIPE + t * ZR, ZR)])

    def zden(r, c):
        for k in range(D // 16):
            den2d[r, pl.ds(k * 16, 16)] = zero16
        return c
    lax.fori_loop(0, NP // D, zden, 0)
    plsc.subcore_barrier()

    pltpu.sync_copy(attn_hbm, attn_v)
    a_sl = tuple(attn_v[pl.ds(k * 16, 16)] for k in range(D // 16))

    def edge_body(e, c):
        acc = zero16
        for k in range(D // 16):
            uu = u_buf[e, pl.ds(k * 16, 16)]
            vv = v_buf[e, pl.ds(k * 16, 16)]
            x = uu + vv
            t = jnp.maximum(x, 0.2 * x)
            acc = acc + t * a_sl[k]
        for k in (8, 4, 2, 1):
            acc = acc + acc.at[iota16 ^ k].get(mode="promise_in_bounds",
                                               unique_indices=True)
        wv = jnp.exp(acc)
        for k in range(D // 16):
            u_buf[e, pl.ds(k * 16, 16)] = u_buf[e, pl.ds(k * 16, 16)] * wv
        wmat[pl.ds(e * 16, 16)] = wv
        return c

    def chunk_body(c, carry):
        base = wid * EW + c * B
        pltpu.sync_copy(src_hbm.at[pl.ds(base, B)], idx_src)
        pltpu.sync_copy(dst_hbm.at[pl.ds(base, B)], idx_dst)
        pltpu.async_copy(up_hbm.at[idx_src], u_buf, sem).wait()
        pltpu.async_copy(v_hbm.at[idx_dst], v_buf, sem).wait()
        lax.fori_loop(0, B, edge_body, 0)
        pltpu.sync_copy(u_buf, acc_sh.at[idx_dst], add=True)
        for g in range(B // 16):
            dv = idx_dst[pl.ds(g * 16, 16)]
            wvec = plsc.load_gather(wmat, [(iota16 + g * 16) * 16])
            ks, ws = plsc.sort_key_val(dv, wvec)
            for d_ in (1, 2, 4, 8):
                idxd = jnp.minimum(iota16 + d_, 15)
                ksh = ks.at[idxd].get(mode="promise_in_bounds")
                wsh = ws.at[idxd].get(mode="promise_in_bounds")
                cond = (ksh == ks) & (iota16 < 16 - d_)
                ws = ws + jnp.where(cond, wsh, 0.0)
            kprev = ks.at[jnp.maximum(iota16 - 1, 0)].get(
                mode="promise_in_bounds")
            first = (iota16 == 0) | (kprev != ks)
            plsc.addupdate_scatter(
                den2d, [lax.shift_right_logical(ks, 7), ks & 127], ws,
                mask=first)
        return carry

    lax.fori_loop(0, NCHUNK, chunk_body, 0)
    for t in range(5):
        idx_src[pl.ds(t * 16, 16)] = NP + t * 16 + iota16
    pltpu.sync_copy(den2d, acc_sh.at[idx_src], add=True)
    plsc.subcore_barrier()
    pltpu.sync_copy(acc_sh.at[pl.ds(sid * STRIPE, STRIPE)],
                    out_hbm.at[cid, pl.ds(sid * STRIPE, STRIPE)])


_sc_edge = pl.kernel(
    _sc_edge_body,
    out_type=jax.ShapeDtypeStruct((NC, ACC_ROWS, D), jnp.float32),
    mesh=plsc.VectorSubcoreMesh(core_axis_name="c", subcore_axis_name="s",
                                num_cores=NC, num_subcores=NS),
    compiler_params=pltpu.CompilerParams(needs_layout_passes=False),
    scratch_types=[
        pltpu.VMEM((B,), jnp.int32),
        pltpu.VMEM((B,), jnp.int32),
        pltpu.VMEM((B, D), jnp.float32),
        pltpu.VMEM((B, D), jnp.float32),
        pltpu.VMEM((B * 16,), jnp.float32),
        pltpu.VMEM((NP // D, D), jnp.float32),
        pltpu.VMEM((D,), jnp.float32),
        pltpu.VMEM((ZR, D), jnp.float32),
        pltpu.VMEM_SHARED((ACC_ROWS, D), jnp.float32),
        pltpu.SemaphoreType.DMA,
    ],
)

_RB = 1000


def kernel(feats, edge_index, W_src, W_dst, attn, bias, R0_W, R0_b, R1_W, R1_b):
    src = edge_index[0].astype(jnp.int32)
    dst = edge_index[1].astype(jnp.int32)
    attn_flat = attn.reshape(D).astype(jnp.float32)

    up, v, s0 = pl.pallas_call(
        _prep_body,
        grid=(N // _RB,),
        in_specs=[
            pl.BlockSpec((_RB, D), lambda i: (i, 0)),
            pl.BlockSpec((D, D), lambda i: (0, 0)),
            pl.BlockSpec((D, D), lambda i: (0, 0)),
            pl.BlockSpec((D, D), lambda i: (0, 0)),
        ],
        out_specs=[
            pl.BlockSpec((_RB, D), lambda i: (i, 0)),
            pl.BlockSpec((_RB, D), lambda i: (i, 0)),
            pl.BlockSpec((_RB, D), lambda i: (i, 0)),
        ],
        out_shape=[
            jax.ShapeDtypeStruct((N, D), jnp.float32),
            jax.ShapeDtypeStruct((N, D), jnp.float32),
            jax.ShapeDtypeStruct((N, D), jnp.float32),
        ],
    )(feats, W_src, W_dst, R0_W)

    acc = _sc_edge(up, v, src, dst, attn_flat)

    den = (acc[0, NP:NP + N // D + 1] + acc[1, NP:NP + N // D + 1])
    den = den.reshape(-1)[:N, None]
    den_b = jnp.broadcast_to(den, (N, D))

    rb = (R0_b + R1_b).reshape(1, D).astype(jnp.float32)
    bias2 = bias.reshape(1, D).astype(jnp.float32)
    _FB = 1024
    out = pl.pallas_call(
        _final_body,
        grid=(NP // _FB,),
        in_specs=[
            pl.BlockSpec((NC, _FB, D), lambda i: (0, i, 0)),
            pl.BlockSpec((_FB, D), lambda i: (i, 0)),
            pl.BlockSpec((_FB, D), lambda i: (i, 0)),
            pl.BlockSpec((1, D), lambda i: (0, 0)),
            pl.BlockSpec((D, D), lambda i: (0, 0)),
            pl.BlockSpec((1, D), lambda i: (0, 0)),
        ],
        out_specs=pl.BlockSpec((_FB, D), lambda i: (i, 0)),
        out_shape=jax.ShapeDtypeStruct((N, D), jnp.float32),
    )(acc[:, :NP], s0, den_b, bias2, R1_W, rb)
    return out
